# baseline (speedup 1.0000x reference)
import functools
import math

import jax
import jax.numpy as jnp
from jax import lax
from jax.experimental import pallas as pl
from jax.experimental.pallas import tpu as pltpu
from jax.experimental.pallas import tpu_sc as plsc

_HP = lax.Precision.HIGHEST

_B, _Q, _S, _T, _D = 8, 4, 64, 512, 128
_BQ = _B * _Q
_STAT_K = 8
_TOK_K = 64
_NT = _BQ * _STAT_K
_RSQD = 1.0 / math.sqrt(_D)


def _prep_body(vl_ref, q_ref, sk_ref, wqs_ref, wqt_ref, wks_ref,
               qt_ref, sw_ref, rid_ref, ridp_ref):
    q2 = q_ref[...]
    qs = lax.dot(q2, wqs_ref[...])
    qt = lax.dot(q2, wqt_ref[...])
    ks = lax.dot(sk_ref[...], wks_ref[...])
    scores = lax.dot_general(qs, ks, (((1,), (1,)), ((), ()))) * _RSQD
    col = lax.broadcasted_iota(jnp.int32, (_BQ, _B * _S), 1)
    row = lax.broadcasted_iota(jnp.int32, (_BQ, _B * _S), 0)
    own = (col // _S) == (row // _Q)
    vlrow = jnp.zeros((_BQ, _B * _S), jnp.int32)
    for b in range(_B):
        vlrow = jnp.where(row // _Q == b, vl_ref[0, b], vlrow)
    valid = (col % _S) < vlrow
    scores = jnp.where(own & valid, scores,
                       jnp.where(own, -1000000.0, -1e30))
    cur = scores
    vals, idxs = [], []
    for _ in range(_STAT_K):
        m = jnp.max(cur, axis=1, keepdims=True)
        i = jnp.min(jnp.where(cur == m, col, 1 << 30), axis=1, keepdims=True)
        vals.append(m)
        idxs.append(i)
        cur = jnp.where(col == i, -3e30, cur)
    vals = jnp.concatenate(vals, axis=1)
    idxs = jnp.concatenate(idxs, axis=1)
    e = jnp.exp(vals - vals[:, :1])
    sw = e / jnp.sum(e, axis=1, keepdims=True)
    qt_ref[...] = qt
    sw_ref[...] = sw
    rid_ref[...] = idxs
    ridp_ref[...] = jnp.concatenate(
        [idxs, jnp.zeros((_BQ, _D - _STAT_K), jnp.int32)], axis=1)


_GB = 64


def _score_body(rid_sm, *refs):
    tkbs = refs[:_GB]
    wkt_ref, qt_ref, o_ref = refs[_GB:]
    wkt = wkt_ref[...]
    for k, tkb in enumerate(tkbs):
        qrow = qt_ref[0, (k // _STAT_K):(k // _STAT_K) + 1]
        ktok = lax.dot(tkb[0], wkt)
        s = lax.dot_general(qrow, ktok,
                            (((1,), (1,)), ((), ()))) * _RSQD
        o_ref[0, k] = s[0]


def _weight_body(s_ref, sw_ref, o_ref):
    s = s_ref[...]
    u = lax.bitcast_convert_type(s, jnp.int32)
    keys = u ^ ((u >> 31) & 0x7FFFFFFF)

    def bsearch(_, lohi):
        lo, hi = lohi
        diff = hi - lo
        mid = lo + (lax.shift_right_logical(diff, 1) + (diff & 1))
        cnt = jnp.sum((keys >= mid).astype(jnp.int32), axis=1, keepdims=True)
        g = cnt >= _TOK_K
        return (jnp.where(g, mid, lo), jnp.where(g, hi, mid - 1))

    kt, _hi = lax.fori_loop(
        0, 32, bsearch,
        (jnp.full((_NT, 1), -(2 ** 31), jnp.int32),
         jnp.full((_NT, 1), 2 ** 31 - 1, jnp.int32)))
    sel = keys >= kt
    m = jnp.max(s, axis=1, keepdims=True)
    ev = jnp.where(sel, jnp.exp(s - m), 0.0)
    z = jnp.sum(ev, axis=1, keepdims=True)
    o_ref[...] = ev * (sw_ref[...] / z)


def _fin_body(acc_ref, wv_ref, wo_ref, out_ref):
    out_ref[...] = lax.dot(lax.dot(acc_ref[...], wv_ref[...], precision=_HP),
                           wo_ref[...], precision=_HP)


_NC = 2
_NS = 16


def _sc_body(w_hbm, va_hbm, rid_hbm, out_hbm,
             rid_v, wrow0, wrow1, selid0, selid1, selw0, selw1,
             vrows0, vrows1, outb, semw0, semw1, semg0, semg1):
    wid = lax.axis_index("s") * _NC + lax.axis_index("c")
    pltpu.sync_copy(rid_hbm.at[wid], rid_v)
    lanes = lax.iota(jnp.int32, 16)
    zf = jnp.zeros((16,), jnp.float32)
    zi = jnp.zeros((16,), jnp.int32)
    wrows = (wrow0, wrow1)
    selids = (selid0, selid1)
    selws = (selw0, selw1)
    vrowss = (vrows0, vrows1)
    semws = (semw0, semw1)
    semgs = (semg0, semg1)

    def jbody(j, acc):
        jf = jnp.full((16,), j, jnp.int32)
        bsv = plsc.load_gather(rid_v, [jf])
        wrow, selid, selw, vrows, sem = wrow0, selid0, selw0, vrows0, semw0
        pltpu.sync_copy(w_hbm.at[wid * _STAT_K + j], wrow)
        for c in range(_TOK_K // 16):
            selw[pl.ds(c * 16, 16)] = zf
            selid[pl.ds(c * 16, 16)] = zi

        def cpass(c, off):
            wv16 = wrow[pl.ds(c * 16, 16)]
            msel = wv16 > 0.0
            pos = off + plsc.cumsum(msel.astype(jnp.int32)) - 1
            okm = msel & (pos < _TOK_K)
            gid = bsv * _T + c * 16 + lanes
            plsc.store_scatter(selid, [pos], gid, mask=okm)
            plsc.store_scatter(selw, [pos], wv16, mask=okm)
            return off + plsc.all_reduce_population_count(msel)

        lax.fori_loop(0, _T // 16, cpass, jnp.zeros((16,), jnp.int32))
        pltpu.async_copy(va_hbm.at[selid], vrows, sem).wait()

        def abody(t2, acc2):
            for t in (t2 * 2, t2 * 2 + 1):
                wv = plsc.load_gather(selw, [jnp.full((16,), t, jnp.int32)])
                acc2 = tuple(acc2[c] + wv * vrows[t, pl.ds(c * 16, 16)]
                             for c in range(8))
            return acc2

        return lax.fori_loop(0, _TOK_K // 2, abody, acc)

    acc = lax.fori_loop(0, _STAT_K, jbody,
                        tuple(jnp.zeros((16,), jnp.float32) for _ in range(8)))
    for c in range(8):
        outb[pl.ds(c * 16, 16)] = acc[c]
    pltpu.sync_copy(outb, out_hbm.at[wid])


@functools.cache
def _sc_main():
    return pl.kernel(
        _sc_body,
        out_type=jax.ShapeDtypeStruct((_BQ, _D), jnp.float32),
        mesh=plsc.VectorSubcoreMesh(core_axis_name="c", subcore_axis_name="s",
                                    num_cores=_NC, num_subcores=_NS),
        compiler_params=pltpu.CompilerParams(needs_layout_passes=False),
        scratch_types=[
            pltpu.VMEM((_D,), jnp.int32),
            pltpu.VMEM((_T,), jnp.float32),
            pltpu.VMEM((_T,), jnp.float32),
            pltpu.VMEM((_TOK_K,), jnp.int32),
            pltpu.VMEM((_TOK_K,), jnp.int32),
            pltpu.VMEM((_D,), jnp.float32),
            pltpu.VMEM((_D,), jnp.float32),
            pltpu.VMEM((_TOK_K, _D), jnp.float32),
            pltpu.VMEM((_TOK_K, _D), jnp.float32),
            pltpu.VMEM((_D,), jnp.float32),
            pltpu.SemaphoreType.DMA,
            pltpu.SemaphoreType.DMA,
            pltpu.SemaphoreType.DMA,
            pltpu.SemaphoreType.DMA,
        ],
    )


def kernel(queries, stat_keys, token_keys, values, stat_valid_lens,
           Wq_stat, Wq_token, Wk_stat, Wk_token, Wv, Wo):
    q2 = queries.reshape(_BQ, _D)
    sk = stat_keys.reshape(_B * _S, _D)
    vaf = values.reshape(_B * _S * _T, _D)
    vl = stat_valid_lens.reshape(1, _B)

    qt, sw, rid, rid_p = pl.pallas_call(
        _prep_body,
        out_shape=[
            jax.ShapeDtypeStruct((_BQ, _D), jnp.float32),
            jax.ShapeDtypeStruct((_BQ, _STAT_K), jnp.float32),
            jax.ShapeDtypeStruct((_BQ, _STAT_K), jnp.int32),
            jax.ShapeDtypeStruct((_BQ, _D), jnp.int32),
        ],
        in_specs=[pl.BlockSpec(memory_space=pltpu.SMEM)] + [pl.BlockSpec()] * 5,
    )(vl, q2, sk, Wq_stat, Wq_token, Wk_stat)

    rid_flat = rid.reshape(_NT)

    _nsteps = _NT // _GB
    _bqper = _GB // _STAT_K
    tk_specs = [
        pl.BlockSpec((1, _T, _D),
                     lambda i, rid_sm, k=k: (rid_sm[i * _GB + k], 0, 0))
        for k in range(_GB)
    ]
    scores = pl.pallas_call(
        _score_body,
        grid_spec=pltpu.PrefetchScalarGridSpec(
            num_scalar_prefetch=1,
            grid=(_nsteps,),
            in_specs=tk_specs + [
                pl.BlockSpec((_D, _D), lambda i, rid_sm: (0, 0)),
                pl.BlockSpec((1, _bqper, _D), lambda i, rid_sm: (i, 0, 0)),
            ],
            out_specs=pl.BlockSpec((1, _GB, _T),
                                   lambda i, rid_sm: (i, 0, 0)),
        ),
        out_shape=jax.ShapeDtypeStruct((_nsteps, _GB, _T), jnp.float32),
    )(rid_flat, *([token_keys] * _GB), Wk_token,
      qt.reshape(_nsteps, _bqper, _D))

    wrows = pl.pallas_call(
        _weight_body,
        out_shape=jax.ShapeDtypeStruct((_NT, _T), jnp.float32),
    )(scores.reshape(_NT, _T), sw.reshape(_NT, 1))

    acc = _sc_main()(wrows, vaf, rid_p)

    out = pl.pallas_call(
        _fin_body,
        out_shape=jax.ShapeDtypeStruct((_BQ, _D), jnp.float32),
    )(acc, Wv, Wo)
    return out.reshape(_B, _Q, _D)


# speedup vs baseline: 1.0024x; 1.0024x over previous
import functools
import math

import jax
import jax.numpy as jnp
from jax import lax
from jax.experimental import pallas as pl
from jax.experimental.pallas import tpu as pltpu
from jax.experimental.pallas import tpu_sc as plsc

_HP = lax.Precision.HIGHEST

_B, _Q, _S, _T, _D = 8, 4, 64, 512, 128
_BQ = _B * _Q
_STAT_K = 8
_TOK_K = 64
_NT = _BQ * _STAT_K
_RSQD = 1.0 / math.sqrt(_D)


def _prep_body(vl_ref, q_ref, sk_ref, wqs_ref, wqt_ref, wks_ref,
               qt_ref, sw_ref, rid_ref, ridp_ref):
    q2 = q_ref[...]
    qs = lax.dot(q2, wqs_ref[...])
    qt = lax.dot(q2, wqt_ref[...])
    ks = lax.dot(sk_ref[...], wks_ref[...])
    scores = lax.dot_general(qs, ks, (((1,), (1,)), ((), ()))) * _RSQD
    col = lax.broadcasted_iota(jnp.int32, (_BQ, _B * _S), 1)
    row = lax.broadcasted_iota(jnp.int32, (_BQ, _B * _S), 0)
    own = (col // _S) == (row // _Q)
    vlrow = jnp.zeros((_BQ, _B * _S), jnp.int32)
    for b in range(_B):
        vlrow = jnp.where(row // _Q == b, vl_ref[0, b], vlrow)
    valid = (col % _S) < vlrow
    scores = jnp.where(own & valid, scores,
                       jnp.where(own, -1000000.0, -1e30))
    cur = scores
    vals, idxs = [], []
    for _ in range(_STAT_K):
        m = jnp.max(cur, axis=1, keepdims=True)
        i = jnp.min(jnp.where(cur == m, col, 1 << 30), axis=1, keepdims=True)
        vals.append(m)
        idxs.append(i)
        cur = jnp.where(col == i, -3e30, cur)
    vals = jnp.concatenate(vals, axis=1)
    idxs = jnp.concatenate(idxs, axis=1)
    e = jnp.exp(vals - vals[:, :1])
    sw = e / jnp.sum(e, axis=1, keepdims=True)
    qt_ref[...] = qt
    sw_ref[...] = sw
    rid_ref[...] = idxs
    ridp_ref[...] = jnp.concatenate(
        [idxs, jnp.zeros((_BQ, _D - _STAT_K), jnp.int32)], axis=1)


_GB = 32


def _score_body(rid_sm, *refs):
    tkbs = refs[:_GB]
    wkt_ref, qt_ref, o_ref = refs[_GB:]
    wkt = wkt_ref[...]
    for k, tkb in enumerate(tkbs):
        qrow = qt_ref[0, (k // _STAT_K):(k // _STAT_K) + 1]
        ktok = lax.dot(tkb[0], wkt)
        s = lax.dot_general(qrow, ktok,
                            (((1,), (1,)), ((), ()))) * _RSQD
        o_ref[0, k] = s[0]


def _weight_body(s_ref, sw_ref, o_ref):
    s = s_ref[...]
    u = lax.bitcast_convert_type(s, jnp.int32)
    keys = u ^ ((u >> 31) & 0x7FFFFFFF)

    def bsearch(_, lohi):
        lo, hi = lohi
        diff = hi - lo
        mid = lo + (lax.shift_right_logical(diff, 1) + (diff & 1))
        cnt = jnp.sum((keys >= mid).astype(jnp.int32), axis=1, keepdims=True)
        g = cnt >= _TOK_K
        return (jnp.where(g, mid, lo), jnp.where(g, hi, mid - 1))

    kt, _hi = lax.fori_loop(
        0, 32, bsearch,
        (jnp.full((_NT, 1), -(2 ** 31), jnp.int32),
         jnp.full((_NT, 1), 2 ** 31 - 1, jnp.int32)))
    sel = keys >= kt
    m = jnp.max(s, axis=1, keepdims=True)
    ev = jnp.where(sel, jnp.exp(s - m), 0.0)
    z = jnp.sum(ev, axis=1, keepdims=True)
    o_ref[...] = ev * (sw_ref[...] / z)


def _fin_body(acc_ref, wv_ref, wo_ref, out_ref):
    out_ref[...] = lax.dot(lax.dot(acc_ref[...], wv_ref[...], precision=_HP),
                           wo_ref[...], precision=_HP)


_NC = 2
_NS = 16


def _sc_body(w_hbm, va_hbm, rid_hbm, out_hbm,
             rid_v, wrow0, wrow1, selid0, selid1, selw0, selw1,
             vrows0, vrows1, outb, semw0, semw1, semg0, semg1):
    wid = lax.axis_index("s") * _NC + lax.axis_index("c")
    pltpu.sync_copy(rid_hbm.at[wid], rid_v)
    lanes = lax.iota(jnp.int32, 16)
    zf = jnp.zeros((16,), jnp.float32)
    zi = jnp.zeros((16,), jnp.int32)
    wrows = (wrow0, wrow1)
    selids = (selid0, selid1)
    selws = (selw0, selw1)
    vrowss = (vrows0, vrows1)
    semws = (semw0, semw1)
    semgs = (semg0, semg1)

    def jbody(j, acc):
        jf = jnp.full((16,), j, jnp.int32)
        bsv = plsc.load_gather(rid_v, [jf])
        wrow, selid, selw, vrows, sem = wrow0, selid0, selw0, vrows0, semw0
        pltpu.sync_copy(w_hbm.at[wid * _STAT_K + j], wrow)
        for c in range(_TOK_K // 16):
            selw[pl.ds(c * 16, 16)] = zf
            selid[pl.ds(c * 16, 16)] = zi

        def cpass(c, off):
            wv16 = wrow[pl.ds(c * 16, 16)]
            msel = wv16 > 0.0
            pos = off + plsc.cumsum(msel.astype(jnp.int32)) - 1
            okm = msel & (pos < _TOK_K)
            gid = bsv * _T + c * 16 + lanes
            plsc.store_scatter(selid, [pos], gid, mask=okm)
            plsc.store_scatter(selw, [pos], wv16, mask=okm)
            return off + plsc.all_reduce_population_count(msel)

        lax.fori_loop(0, _T // 16, cpass, jnp.zeros((16,), jnp.int32))
        pltpu.async_copy(va_hbm.at[selid], vrows, sem).wait()

        def abody(t2, acc2):
            for t in (t2 * 2, t2 * 2 + 1):
                wv = plsc.load_gather(selw, [jnp.full((16,), t, jnp.int32)])
                acc2 = tuple(acc2[c] + wv * vrows[t, pl.ds(c * 16, 16)]
                             for c in range(8))
            return acc2

        return lax.fori_loop(0, _TOK_K // 2, abody, acc)

    acc = lax.fori_loop(0, _STAT_K, jbody,
                        tuple(jnp.zeros((16,), jnp.float32) for _ in range(8)))
    for c in range(8):
        outb[pl.ds(c * 16, 16)] = acc[c]
    pltpu.sync_copy(outb, out_hbm.at[wid])


@functools.cache
def _sc_main():
    return pl.kernel(
        _sc_body,
        out_type=jax.ShapeDtypeStruct((_BQ, _D), jnp.float32),
        mesh=plsc.VectorSubcoreMesh(core_axis_name="c", subcore_axis_name="s",
                                    num_cores=_NC, num_subcores=_NS),
        compiler_params=pltpu.CompilerParams(needs_layout_passes=False),
        scratch_types=[
            pltpu.VMEM((_D,), jnp.int32),
            pltpu.VMEM((_T,), jnp.float32),
            pltpu.VMEM((_T,), jnp.float32),
            pltpu.VMEM((_TOK_K,), jnp.int32),
            pltpu.VMEM((_TOK_K,), jnp.int32),
            pltpu.VMEM((_D,), jnp.float32),
            pltpu.VMEM((_D,), jnp.float32),
            pltpu.VMEM((_TOK_K, _D), jnp.float32),
            pltpu.VMEM((_TOK_K, _D), jnp.float32),
            pltpu.VMEM((_D,), jnp.float32),
            pltpu.SemaphoreType.DMA,
            pltpu.SemaphoreType.DMA,
            pltpu.SemaphoreType.DMA,
            pltpu.SemaphoreType.DMA,
        ],
    )


def kernel(queries, stat_keys, token_keys, values, stat_valid_lens,
           Wq_stat, Wq_token, Wk_stat, Wk_token, Wv, Wo):
    q2 = queries.reshape(_BQ, _D)
    sk = stat_keys.reshape(_B * _S, _D)
    vaf = values.reshape(_B * _S * _T, _D)
    vl = stat_valid_lens.reshape(1, _B)

    qt, sw, rid, rid_p = pl.pallas_call(
        _prep_body,
        out_shape=[
            jax.ShapeDtypeStruct((_BQ, _D), jnp.float32),
            jax.ShapeDtypeStruct((_BQ, _STAT_K), jnp.float32),
            jax.ShapeDtypeStruct((_BQ, _STAT_K), jnp.int32),
            jax.ShapeDtypeStruct((_BQ, _D), jnp.int32),
        ],
        in_specs=[pl.BlockSpec(memory_space=pltpu.SMEM)] + [pl.BlockSpec()] * 5,
    )(vl, q2, sk, Wq_stat, Wq_token, Wk_stat)

    rid_flat = rid.reshape(_NT)

    _nsteps = _NT // _GB
    _bqper = _GB // _STAT_K
    tk_specs = [
        pl.BlockSpec((1, _T, _D),
                     lambda i, rid_sm, k=k: (rid_sm[i * _GB + k], 0, 0))
        for k in range(_GB)
    ]
    scores = pl.pallas_call(
        _score_body,
        grid_spec=pltpu.PrefetchScalarGridSpec(
            num_scalar_prefetch=1,
            grid=(_nsteps,),
            in_specs=tk_specs + [
                pl.BlockSpec((_D, _D), lambda i, rid_sm: (0, 0)),
                pl.BlockSpec((1, _bqper, _D), lambda i, rid_sm: (i, 0, 0)),
            ],
            out_specs=pl.BlockSpec((1, _GB, _T),
                                   lambda i, rid_sm: (i, 0, 0)),
        ),
        out_shape=jax.ShapeDtypeStruct((_nsteps, _GB, _T), jnp.float32),
    )(rid_flat, *([token_keys] * _GB), Wk_token,
      qt.reshape(_nsteps, _bqper, _D))

    wrows = pl.pallas_call(
        _weight_body,
        out_shape=jax.ShapeDtypeStruct((_NT, _T), jnp.float32),
    )(scores.reshape(_NT, _T), sw.reshape(_NT, 1))

    acc = _sc_main()(wrows, vaf, rid_p)

    out = pl.pallas_call(
        _fin_body,
        out_shape=jax.ShapeDtypeStruct((_BQ, _D), jnp.float32),
    )(acc, Wv, Wo)
    return out.reshape(_B, _Q, _D)
